# Initial kernel scaffold; baseline (speedup 1.0000x reference)
import jax
import jax.numpy as jnp
from jax.experimental import pallas as pl


def kernel(x, table):
    raise NotImplementedError("write your pallas kernel here")


# speedup vs baseline: 2.4162x; 2.4162x over previous
import functools

import jax
import jax.numpy as jnp
from jax import lax
from jax.experimental import pallas as pl
from jax.experimental.pallas import tpu as pltpu
from jax.experimental.pallas import tpu_sc as plsc

B_TOTAL = 4 * 8192
D_MODEL = 512
NUM_WORKERS = 32
BPW = B_TOTAL // NUM_WORKERS
CHUNK = 128
NCHUNKS = BPW // CHUNK

_mesh = plsc.VectorSubcoreMesh(core_axis_name="c", subcore_axis_name="s")


@functools.partial(
    pl.kernel,
    mesh=_mesh,
    out_type=jax.ShapeDtypeStruct((B_TOTAL, D_MODEL), jnp.float32),
    scratch_types=[
        pltpu.VMEM((BPW,), jnp.int32),
        pltpu.VMEM((CHUNK, D_MODEL), jnp.float32),
        pltpu.SemaphoreType.DMA,
    ],
)
def _embed_gather(idx_hbm, table_hbm, out_hbm, idx_v, rows_v, sem):
    wid = lax.axis_index("s") * 2 + lax.axis_index("c")
    base = wid * BPW
    pltpu.sync_copy(idx_hbm.at[pl.ds(base, BPW)], idx_v)
    for c in range(NCHUNKS):
        idx_slice = idx_v.at[pl.ds(c * CHUNK, CHUNK)]
        pltpu.async_copy(table_hbm.at[idx_slice], rows_v, sem).wait()
        pltpu.sync_copy(rows_v, out_hbm.at[pl.ds(base + c * CHUNK, CHUNK)])


def kernel(x, table):
    out = _embed_gather(x.reshape(-1).astype(jnp.int32), table)
    return out.reshape(x.shape + (D_MODEL,))


# speedup vs baseline: 2.5983x; 1.0753x over previous
import functools

import jax
import jax.numpy as jnp
from jax import lax
from jax.experimental import pallas as pl
from jax.experimental.pallas import tpu as pltpu
from jax.experimental.pallas import tpu_sc as plsc

B_TOTAL = 4 * 8192
D_MODEL = 512
NUM_WORKERS = 32
BPW = B_TOTAL // NUM_WORKERS
CHUNK = 64
NCHUNKS = BPW // CHUNK
NBUF = 3

_mesh = plsc.VectorSubcoreMesh(core_axis_name="c", subcore_axis_name="s")


@functools.partial(
    pl.kernel,
    mesh=_mesh,
    out_type=jax.ShapeDtypeStruct((B_TOTAL, D_MODEL), jnp.float32),
    scratch_types=[
        pltpu.VMEM((BPW,), jnp.int32),
        *[pltpu.VMEM((CHUNK, D_MODEL), jnp.float32) for _ in range(NBUF)],
        *[pltpu.SemaphoreType.DMA for _ in range(2 * NBUF)],
    ],
)
def _embed_gather(idx_hbm, table_hbm, out_hbm, idx_v, *bufs_and_sems):
    bufs = bufs_and_sems[:NBUF]
    gsems = bufs_and_sems[NBUF:2 * NBUF]
    wsems = bufs_and_sems[2 * NBUF:]
    wid = lax.axis_index("s") * 2 + lax.axis_index("c")
    base = wid * BPW
    pltpu.sync_copy(idx_hbm.at[pl.ds(base, BPW)], idx_v)

    def start_gather(c):
        b = c % NBUF
        idx_slice = idx_v.at[pl.ds(c * CHUNK, CHUNK)]
        return pltpu.async_copy(table_hbm.at[idx_slice], bufs[b], gsems[b])

    def start_write(c):
        b = c % NBUF
        dst = out_hbm.at[pl.ds(base + c * CHUNK, CHUNK)]
        return pltpu.async_copy(bufs[b], dst, wsems[b])

    gathers = [None] * NCHUNKS
    writes = [None] * NCHUNKS
    for c in range(NBUF - 1):
        gathers[c] = start_gather(c)
    for c in range(NCHUNKS):
        if c > 0:
            writes[c - 1].wait()
        g = c + NBUF - 1
        if g < NCHUNKS:
            gathers[g] = start_gather(g)
        gathers[c].wait()
        writes[c] = start_write(c)
    writes[NCHUNKS - 1].wait()


def kernel(x, table):
    out = _embed_gather(x.reshape(-1).astype(jnp.int32), table)
    return out.reshape(x.shape + (D_MODEL,))
